# baseline (speedup 1.0000x reference)
import jax
import jax.numpy as jnp
from jax.experimental import pallas as pl
from jax.experimental.pallas import tpu as pltpu

_BT = 256


def _body(x_ref, pos_ref, scale_ref, o_ref):
    o_ref[...] = x_ref[...] + scale_ref[0] * pos_ref[...]


def kernel(x, pos_weight, scale):
    b, t, d = x.shape
    nt = t // _BT
    grid = (nt, b)
    return pl.pallas_call(
        _body,
        grid=grid,
        in_specs=[
            pl.BlockSpec((1, _BT, d), lambda i, j: (j, i, 0)),
            pl.BlockSpec((_BT, d), lambda i, j: (i, 0)),
            pl.BlockSpec(memory_space=pltpu.SMEM),
        ],
        out_specs=pl.BlockSpec((1, _BT, d), lambda i, j: (j, i, 0)),
        out_shape=jax.ShapeDtypeStruct((b, t, d), x.dtype),
    )(x, pos_weight[:t], scale)


# speedup vs baseline: 1.4560x; 1.4560x over previous
import jax
import jax.numpy as jnp
from jax.experimental import pallas as pl
from jax.experimental.pallas import tpu as pltpu

_BT = 1024


def _body(x_ref, pos_ref, scale_ref, o_ref):
    o_ref[...] = x_ref[...] + scale_ref[0] * pos_ref[...]


def kernel(x, pos_weight, scale):
    b, t, d = x.shape
    nt = t // _BT
    grid = (nt, b)
    return pl.pallas_call(
        _body,
        grid=grid,
        in_specs=[
            pl.BlockSpec((1, _BT, d), lambda i, j: (j, i, 0)),
            pl.BlockSpec((_BT, d), lambda i, j: (i, 0)),
            pl.BlockSpec(memory_space=pltpu.SMEM),
        ],
        out_specs=pl.BlockSpec((1, _BT, d), lambda i, j: (j, i, 0)),
        out_shape=jax.ShapeDtypeStruct((b, t, d), x.dtype),
    )(x, pos_weight[:t], scale)


# speedup vs baseline: 1.5365x; 1.0553x over previous
import jax
import jax.numpy as jnp
from jax.experimental import pallas as pl
from jax.experimental.pallas import tpu as pltpu

_BT = 2048


def _body(x_ref, pos_ref, scale_ref, o_ref):
    o_ref[...] = x_ref[...] + scale_ref[0] * pos_ref[...]


def kernel(x, pos_weight, scale):
    b, t, d = x.shape
    nt = t // _BT
    grid = (nt, b)
    return pl.pallas_call(
        _body,
        grid=grid,
        in_specs=[
            pl.BlockSpec((1, _BT, d), lambda i, j: (j, i, 0)),
            pl.BlockSpec((_BT, d), lambda i, j: (i, 0)),
            pl.BlockSpec(memory_space=pltpu.SMEM),
        ],
        out_specs=pl.BlockSpec((1, _BT, d), lambda i, j: (j, i, 0)),
        out_shape=jax.ShapeDtypeStruct((b, t, d), x.dtype),
    )(x, pos_weight[:t], scale)
